# baseline (speedup 1.0000x reference)
import jax
import jax.numpy as jnp
from jax import lax
from jax.experimental import pallas as pl
from jax.experimental.pallas import tpu as pltpu
from jax.experimental.pallas import tpu_sc as plsc

V = 1000000
D = 64
B = 16384
K = 5
NC = 2
NS = 16
NW = NC * NS
PER_W = B // NW
C = 32
NCHUNK = PER_W // C
R = 1 + K
TRC = 512
NTR = (V + TRC - 1) // TRC


def _softplus(u):
    t = jnp.exp(-jnp.abs(u))
    s = t / (t + 2.0)
    p = s * s
    ser = s * (1.0 + p * (1.0 / 3.0 + p * (1.0 / 5.0 + p * (1.0 / 7.0 + p * (1.0 / 9.0)))))
    return jnp.maximum(u, 0.0) + 2.0 * ser


def _tr_body(inb, outb):
    x = inb[...]
    eye = jax.lax.broadcasted_iota(jnp.int32, (D, D), 0) == \
        jax.lax.broadcasted_iota(jnp.int32, (D, D), 1)
    y = jax.lax.dot_general(x, eye.astype(jnp.float32), (((0,), (0,)), ((), ())),
                            preferred_element_type=jnp.float32)
    outb[:, pl.ds(0, D)] = y
    outb[:, pl.ds(D, D)] = y


def _repack(tablet):
    return pl.pallas_call(
        _tr_body,
        grid=(NTR,),
        in_specs=[pl.BlockSpec((D, TRC), lambda j: (0, j))],
        out_specs=pl.BlockSpec((TRC, 2 * D), lambda j: (j, 0)),
        out_shape=jax.ShapeDtypeStruct((V, 2 * D), jnp.float32),
    )(tablet)


def _body(center_hbm, context_hbm, cids_hbm, xids_hbm, negf_hbm, out_hbm,
          cidx, xidx, nidx, crow, xrow, nrow, stage, accv, sem0, sem1):
    wid = lax.axis_index("s") * NC + lax.axis_index("c")
    base = wid * PER_W

    pltpu.sync_copy(cids_hbm.at[pl.ds(base, PER_W)], cidx)
    pltpu.sync_copy(xids_hbm.at[pl.ds(base, PER_W)], xidx)
    pltpu.sync_copy(negf_hbm.at[pl.ds(base * K, PER_W * K)], nidx)

    sems = (sem0, sem1)

    def fire(j, s):
        jc = j * C
        sem = sems[s]
        cps = [pltpu.async_copy(center_hbm.at[cidx.at[pl.ds(jc, C)]],
                                crow.at[s], sem),
               pltpu.async_copy(context_hbm.at[xidx.at[pl.ds(jc, C)]],
                                xrow.at[s], sem),
               pltpu.async_copy(context_hbm.at[nidx.at[pl.ds(jc * K, 128)]],
                                nrow.at[s, pl.ds(0, 128)], sem),
               pltpu.async_copy(context_hbm.at[nidx.at[pl.ds(jc * K + 128, 32)]],
                                nrow.at[s, pl.ds(128, 32)], sem)]
        return cps

    def compute(s):
        @plsc.parallel_loop(0, C, unroll=2)
        def _(e):
            c0 = crow[s, e, pl.ds(0, 16)]
            c1 = crow[s, e, pl.ds(16, 16)]
            c2 = crow[s, e, pl.ds(32, 16)]
            c3 = crow[s, e, pl.ds(48, 16)]
            x0 = xrow[s, e, pl.ds(0, 16)]
            x1 = xrow[s, e, pl.ds(16, 16)]
            x2 = xrow[s, e, pl.ds(32, 16)]
            x3 = xrow[s, e, pl.ds(48, 16)]
            pos = c0 * x0 + c1 * x1 + c2 * x2 + c3 * x3
            base_s = e * R * 16
            stage[pl.ds(base_s, 16)] = plsc.cumsum(-pos)
            for k in range(K):
                n0 = nrow[s, e * K + k, pl.ds(0, 16)]
                n1 = nrow[s, e * K + k, pl.ds(16, 16)]
                n2 = nrow[s, e * K + k, pl.ds(32, 16)]
                n3 = nrow[s, e * K + k, pl.ds(48, 16)]
                neg = n0 * c0 + n1 * c1 + n2 * c2 + n3 * c3
                stage[pl.ds(base_s + (1 + k) * 16, 16)] = plsc.cumsum(neg)

    lane = lax.iota(jnp.int32, 16)

    def epilogue(acc):
        @plsc.parallel_loop(0, R * C // 16, unroll=2, carry=acc)
        def acc_out(g, a):
            idx = lane * 16 + (g * 256 + 15)
            v = plsc.load_gather(stage, [idx])
            u = jnp.clip(v, -10.0, 10.0)
            return a + _softplus(u)
        return acc_out

    acc = jnp.zeros((16,), jnp.float32)
    prev = fire(0, 0)
    for j in range(NCHUNK):
        nxt = fire(j + 1, (j + 1) % 2) if j + 1 < NCHUNK else []
        for cp in prev:
            cp.wait()
        compute(j % 2)
        acc = epilogue(acc)
        prev = nxt

    accv[...] = acc
    pltpu.sync_copy(accv, out_hbm.at[wid])


@jax.jit
def _sc_loss(center_table, context_table, center_ids, context_ids, negf):
    centerp = _repack(center_table.T)
    contextp = _repack(context_table.T)
    mesh = plsc.VectorSubcoreMesh(core_axis_name="c", subcore_axis_name="s")
    f = pl.kernel(
        _body,
        out_type=jax.ShapeDtypeStruct((NW, 16), jnp.float32),
        mesh=mesh,
        compiler_params=pltpu.CompilerParams(
            needs_layout_passes=False, use_tc_tiling_on_sc=True),
        scratch_types=[
            pltpu.VMEM((PER_W,), jnp.int32),
            pltpu.VMEM((PER_W,), jnp.int32),
            pltpu.VMEM((K * PER_W,), jnp.int32),
            pltpu.VMEM((2, C, 2 * D), jnp.float32),
            pltpu.VMEM((2, C, 2 * D), jnp.float32),
            pltpu.VMEM((2, C * K, 2 * D), jnp.float32),
            pltpu.VMEM((R * C * 16,), jnp.float32),
            pltpu.VMEM((16,), jnp.float32),
            pltpu.SemaphoreType.DMA,
            pltpu.SemaphoreType.DMA,
        ],
    )
    return f(centerp, contextp, center_ids, context_ids, negf)


def kernel(center_table, context_table, center_ids, context_ids, neg_context_ids):
    negf = neg_context_ids.reshape(-1)
    partials = _sc_loss(center_table, context_table, center_ids,
                        context_ids, negf)
    return jnp.sum(partials) / B


# speedup vs baseline: 4.4810x; 4.4810x over previous
import jax
import jax.numpy as jnp
from jax import lax
from jax.experimental import pallas as pl
from jax.experimental.pallas import tpu as pltpu
from jax.experimental.pallas import tpu_sc as plsc

V = 1000000
D = 64
B = 16384
K = 5
NC = 2
NS = 16
NW = NC * NS
PER_W = B // NW
C = 32
NCHUNK = PER_W // C
R = 1 + K
TRC = 8192
NTR = (V + TRC - 1) // TRC


def _softplus(u):
    t = jnp.exp(-jnp.abs(u))
    s = t / (t + 2.0)
    p = s * s
    ser = s * (1.0 + p * (1.0 / 3.0 + p * (1.0 / 5.0 + p * (1.0 / 7.0 + p * (1.0 / 9.0)))))
    return jnp.maximum(u, 0.0) + 2.0 * ser


def _tr_body(inb, outb):
    x = inb[...]
    eye = jax.lax.broadcasted_iota(jnp.int32, (D, D), 0) == \
        jax.lax.broadcasted_iota(jnp.int32, (D, D), 1)
    y = jax.lax.dot_general(x, eye.astype(jnp.float32), (((0,), (0,)), ((), ())),
                            preferred_element_type=jnp.float32)
    outb[:, pl.ds(0, D)] = y


def _repack(tablet):
    return pl.pallas_call(
        _tr_body,
        grid=(NTR,),
        in_specs=[pl.BlockSpec((D, TRC), lambda j: (0, j))],
        out_specs=pl.BlockSpec((TRC, 2 * D), lambda j: (j, 0)),
        out_shape=jax.ShapeDtypeStruct((V, 2 * D), jnp.float32),
    )(tablet)


def _body(center_hbm, context_hbm, cids_hbm, xids_hbm, negf_hbm, out_hbm,
          cidx, xidx, nidx, crow, xrow, nrow, stage, accv, sem0, sem1):
    wid = lax.axis_index("s") * NC + lax.axis_index("c")
    base = wid * PER_W

    pltpu.sync_copy(cids_hbm.at[pl.ds(base, PER_W)], cidx)
    pltpu.sync_copy(xids_hbm.at[pl.ds(base, PER_W)], xidx)
    pltpu.sync_copy(negf_hbm.at[pl.ds(base * K, PER_W * K)], nidx)

    sems = (sem0, sem1)

    def fire(j, s):
        jc = j * C
        sem = sems[s]
        cps = [pltpu.async_copy(center_hbm.at[cidx.at[pl.ds(jc, C)]],
                                crow.at[s], sem),
               pltpu.async_copy(context_hbm.at[xidx.at[pl.ds(jc, C)]],
                                xrow.at[s], sem),
               pltpu.async_copy(context_hbm.at[nidx.at[pl.ds(jc * K, 128)]],
                                nrow.at[s, pl.ds(0, 128)], sem),
               pltpu.async_copy(context_hbm.at[nidx.at[pl.ds(jc * K + 128, 32)]],
                                nrow.at[s, pl.ds(128, 32)], sem)]
        return cps

    def compute(s):
        @plsc.parallel_loop(0, C, unroll=2)
        def _(e):
            c0 = crow[s, e, pl.ds(0, 16)]
            c1 = crow[s, e, pl.ds(16, 16)]
            c2 = crow[s, e, pl.ds(32, 16)]
            c3 = crow[s, e, pl.ds(48, 16)]
            x0 = xrow[s, e, pl.ds(0, 16)]
            x1 = xrow[s, e, pl.ds(16, 16)]
            x2 = xrow[s, e, pl.ds(32, 16)]
            x3 = xrow[s, e, pl.ds(48, 16)]
            pos = c0 * x0 + c1 * x1 + c2 * x2 + c3 * x3
            base_s = e * R * 16
            stage[pl.ds(base_s, 16)] = plsc.cumsum(-pos)
            for k in range(K):
                n0 = nrow[s, e * K + k, pl.ds(0, 16)]
                n1 = nrow[s, e * K + k, pl.ds(16, 16)]
                n2 = nrow[s, e * K + k, pl.ds(32, 16)]
                n3 = nrow[s, e * K + k, pl.ds(48, 16)]
                neg = n0 * c0 + n1 * c1 + n2 * c2 + n3 * c3
                stage[pl.ds(base_s + (1 + k) * 16, 16)] = plsc.cumsum(neg)

    lane = lax.iota(jnp.int32, 16)

    def epilogue(acc):
        @plsc.parallel_loop(0, R * C // 16, unroll=2, carry=acc)
        def acc_out(g, a):
            idx = lane * 16 + (g * 256 + 15)
            v = plsc.load_gather(stage, [idx])
            u = jnp.clip(v, -10.0, 10.0)
            return a + _softplus(u)
        return acc_out

    acc = jnp.zeros((16,), jnp.float32)
    prev = fire(0, 0)
    for j in range(NCHUNK):
        nxt = fire(j + 1, (j + 1) % 2) if j + 1 < NCHUNK else []
        for cp in prev:
            cp.wait()
        compute(j % 2)
        acc = epilogue(acc)
        prev = nxt

    accv[...] = acc
    pltpu.sync_copy(accv, out_hbm.at[wid])


@jax.jit
def _sc_loss(center_table, context_table, center_ids, context_ids, negf):
    centerp = _repack(center_table.T)
    contextp = _repack(context_table.T)
    mesh = plsc.VectorSubcoreMesh(core_axis_name="c", subcore_axis_name="s")
    f = pl.kernel(
        _body,
        out_type=jax.ShapeDtypeStruct((NW, 16), jnp.float32),
        mesh=mesh,
        compiler_params=pltpu.CompilerParams(
            needs_layout_passes=False, use_tc_tiling_on_sc=True),
        scratch_types=[
            pltpu.VMEM((PER_W,), jnp.int32),
            pltpu.VMEM((PER_W,), jnp.int32),
            pltpu.VMEM((K * PER_W,), jnp.int32),
            pltpu.VMEM((2, C, 2 * D), jnp.float32),
            pltpu.VMEM((2, C, 2 * D), jnp.float32),
            pltpu.VMEM((2, C * K, 2 * D), jnp.float32),
            pltpu.VMEM((R * C * 16,), jnp.float32),
            pltpu.VMEM((16,), jnp.float32),
            pltpu.SemaphoreType.DMA,
            pltpu.SemaphoreType.DMA,
        ],
    )
    return f(centerp, contextp, center_ids, context_ids, negf)


def kernel(center_table, context_table, center_ids, context_ids, neg_context_ids):
    negf = neg_context_ids.reshape(-1)
    partials = _sc_loss(center_table, context_table, center_ids,
                        context_ids, negf)
    return jnp.sum(partials) / B


# speedup vs baseline: 4.6704x; 1.0423x over previous
import jax
import jax.numpy as jnp
from jax import lax
from jax.experimental import pallas as pl
from jax.experimental.pallas import tpu as pltpu
from jax.experimental.pallas import tpu_sc as plsc

V = 1000000
D = 64
B = 16384
K = 5
NC = 2
NS = 16
NW = NC * NS
PER_W = B // NW
C = 64
NCHUNK = PER_W // C
R = 1 + K
TRC = 8192
NTR = (V + TRC - 1) // TRC


def _softplus(u):
    t = jnp.exp(-jnp.abs(u))
    s = t / (t + 2.0)
    p = s * s
    ser = s * (1.0 + p * (1.0 / 3.0 + p * (1.0 / 5.0 + p * (1.0 / 7.0 + p * (1.0 / 9.0)))))
    return jnp.maximum(u, 0.0) + 2.0 * ser


PACK = 1 << 19
TRC2 = 4096
NCB = (V + TRC2 - 1) // TRC2


def _mxu_t(x):
    eye = jax.lax.broadcasted_iota(jnp.int32, (D, D), 0) == \
        jax.lax.broadcasted_iota(jnp.int32, (D, D), 1)
    return jax.lax.dot_general(x, eye.astype(jnp.float32),
                               (((0,), (0,)), ((), ())),
                               preferred_element_type=jnp.float32)


def _tr_body(in1, in2, outb):
    outb[:, pl.ds(0, D)] = _mxu_t(in1[...])
    outb[:, pl.ds(D, D)] = _mxu_t(in2[...])


def _repack(tablet):
    return pl.pallas_call(
        _tr_body,
        grid=(PACK // TRC2,),
        in_specs=[
            pl.BlockSpec((D, TRC2), lambda j: (0, j)),
            pl.BlockSpec((D, TRC2),
                         lambda j: (0, jnp.minimum(j + PACK // TRC2, NCB - 1))),
        ],
        out_specs=pl.BlockSpec((TRC2, 2 * D), lambda j: (j, 0)),
        out_shape=jax.ShapeDtypeStruct((PACK, 2 * D), jnp.float32),
    )(tablet, tablet)


def _body(center_hbm, context_hbm, cids_hbm, xids_hbm, negf_hbm, out_hbm,
          cidx, xidx, nidx, crow, xrow, nrow, stage, accv, sem0, sem1):
    wid = lax.axis_index("s") * NC + lax.axis_index("c")
    base = wid * PER_W

    pltpu.sync_copy(cids_hbm.at[pl.ds(base, PER_W)], cidx)
    pltpu.sync_copy(xids_hbm.at[pl.ds(base, PER_W)], xidx)
    pltpu.sync_copy(negf_hbm.at[pl.ds(base * K, PER_W * K)], nidx)

    def xform(v):
        return ((v & (PACK - 1)) << 1) | jax.lax.shift_right_logical(v, 19)

    @pl.loop(0, PER_W // 16)
    def _(t):
        sl = pl.ds(t * 16, 16)
        cidx[sl] = xform(cidx[sl])
        xidx[sl] = xform(xidx[sl])

    @pl.loop(0, PER_W * K // 16)
    def _(t):
        sl = pl.ds(t * 16, 16)
        nidx[sl] = xform(nidx[sl])

    sems = (sem0, sem1)

    def fire(j, s):
        jc = j * C
        sem = sems[s]
        cps = [pltpu.async_copy(center_hbm.at[cidx.at[pl.ds(jc, C)]],
                                crow.at[s], sem),
               pltpu.async_copy(context_hbm.at[xidx.at[pl.ds(jc, C)]],
                                xrow.at[s], sem)]
        for off, ln in ((0, 128), (128, 128), (256, 64)):
            cps.append(pltpu.async_copy(
                context_hbm.at[nidx.at[pl.ds(jc * K + off, ln)]],
                nrow.at[s, pl.ds(off, ln)], sem))
        return cps

    def compute(s):
        @plsc.parallel_loop(0, C, unroll=2)
        def _(e):
            c0 = crow[s, e, pl.ds(0, 16)]
            c1 = crow[s, e, pl.ds(16, 16)]
            c2 = crow[s, e, pl.ds(32, 16)]
            c3 = crow[s, e, pl.ds(48, 16)]
            x0 = xrow[s, e, pl.ds(0, 16)]
            x1 = xrow[s, e, pl.ds(16, 16)]
            x2 = xrow[s, e, pl.ds(32, 16)]
            x3 = xrow[s, e, pl.ds(48, 16)]
            pos = c0 * x0 + c1 * x1 + c2 * x2 + c3 * x3
            base_s = e * R * 16
            stage[pl.ds(base_s, 16)] = plsc.cumsum(-pos)
            for k in range(K):
                n0 = nrow[s, e * K + k, pl.ds(0, 16)]
                n1 = nrow[s, e * K + k, pl.ds(16, 16)]
                n2 = nrow[s, e * K + k, pl.ds(32, 16)]
                n3 = nrow[s, e * K + k, pl.ds(48, 16)]
                neg = n0 * c0 + n1 * c1 + n2 * c2 + n3 * c3
                stage[pl.ds(base_s + (1 + k) * 16, 16)] = plsc.cumsum(neg)

    lane = lax.iota(jnp.int32, 16)

    def epilogue(acc):
        @plsc.parallel_loop(0, R * C // 16, unroll=2, carry=acc)
        def acc_out(g, a):
            idx = lane * 16 + (g * 256 + 15)
            v = plsc.load_gather(stage, [idx])
            u = jnp.clip(v, -10.0, 10.0)
            return a + _softplus(u)
        return acc_out

    acc = jnp.zeros((16,), jnp.float32)
    prev = fire(0, 0)
    for j in range(NCHUNK):
        nxt = fire(j + 1, (j + 1) % 2) if j + 1 < NCHUNK else []
        for cp in prev:
            cp.wait()
        compute(j % 2)
        acc = epilogue(acc)
        prev = nxt

    accv[...] = acc
    pltpu.sync_copy(accv, out_hbm.at[wid])


@jax.jit
def _sc_loss(center_table, context_table, center_ids, context_ids, negf):
    centerp = _repack(center_table.T).reshape(2 * PACK, D)
    contextp = _repack(context_table.T).reshape(2 * PACK, D)
    mesh = plsc.VectorSubcoreMesh(core_axis_name="c", subcore_axis_name="s")
    f = pl.kernel(
        _body,
        out_type=jax.ShapeDtypeStruct((NW, 16), jnp.float32),
        mesh=mesh,
        compiler_params=pltpu.CompilerParams(
            needs_layout_passes=False, use_tc_tiling_on_sc=False),
        scratch_types=[
            pltpu.VMEM((PER_W,), jnp.int32),
            pltpu.VMEM((PER_W,), jnp.int32),
            pltpu.VMEM((K * PER_W,), jnp.int32),
            pltpu.VMEM((2, C, D), jnp.float32),
            pltpu.VMEM((2, C, D), jnp.float32),
            pltpu.VMEM((2, C * K, D), jnp.float32),
            pltpu.VMEM((R * C * 16,), jnp.float32),
            pltpu.VMEM((16,), jnp.float32),
            pltpu.SemaphoreType.DMA,
            pltpu.SemaphoreType.DMA,
        ],
    )
    return f(centerp, contextp, center_ids, context_ids, negf)


def kernel(center_table, context_table, center_ids, context_ids, neg_context_ids):
    negf = neg_context_ids.reshape(-1)
    partials = _sc_loss(center_table, context_table, center_ids,
                        context_ids, negf)
    return jnp.sum(partials) / B


# speedup vs baseline: 5.3055x; 1.1360x over previous
import jax
import jax.numpy as jnp
from jax import lax
from jax.experimental import pallas as pl
from jax.experimental.pallas import tpu as pltpu
from jax.experimental.pallas import tpu_sc as plsc

V = 1000000
D = 64
B = 16384
K = 5
NC = 2
NS = 16
NW = NC * NS
PER_W = B // NW
C = 64
NCHUNK = PER_W // C
R = 1 + K
TRC = 8192
NTR = (V + TRC - 1) // TRC


def _softplus(u):
    t = jnp.exp(-jnp.abs(u))
    s = t / (t + 2.0)
    p = s * s
    ser = s * (1.0 + p * (1.0 / 3.0 + p * (1.0 / 5.0 + p * (1.0 / 7.0 + p * (1.0 / 9.0)))))
    return jnp.maximum(u, 0.0) + 2.0 * ser


PACK = 1 << 19
TRC2 = 8192
NCB = (V + TRC2 - 1) // TRC2


def _mxu_t(x):
    return jnp.transpose(x)


def _tr_body(in1, in2, outb):
    outb[:, pl.ds(0, D)] = _mxu_t(in1[...])
    outb[:, pl.ds(D, D)] = _mxu_t(in2[...])


def _repack(tablet):
    return pl.pallas_call(
        _tr_body,
        grid=(PACK // TRC2,),
        in_specs=[
            pl.BlockSpec((D, TRC2), lambda j: (0, j)),
            pl.BlockSpec((D, TRC2),
                         lambda j: (0, jnp.minimum(j + PACK // TRC2, NCB - 1))),
        ],
        out_specs=pl.BlockSpec((TRC2, 2 * D), lambda j: (j, 0)),
        out_shape=jax.ShapeDtypeStruct((PACK, 2 * D), jnp.float32),
    )(tablet, tablet)


def _body(center_hbm, context_hbm, cids_hbm, xids_hbm, negf_hbm, out_hbm,
          cidx, xidx, nidx, crow, xrow, nrow, stage, accv, sem0, sem1):
    wid = lax.axis_index("s") * NC + lax.axis_index("c")
    base = wid * PER_W

    pltpu.sync_copy(cids_hbm.at[pl.ds(base, PER_W)], cidx)
    pltpu.sync_copy(xids_hbm.at[pl.ds(base, PER_W)], xidx)
    pltpu.sync_copy(negf_hbm.at[pl.ds(base * K, PER_W * K)], nidx)

    def xform(v):
        return ((v & (PACK - 1)) << 1) | jax.lax.shift_right_logical(v, 19)

    @pl.loop(0, PER_W // 16)
    def _(t):
        sl = pl.ds(t * 16, 16)
        cidx[sl] = xform(cidx[sl])
        xidx[sl] = xform(xidx[sl])

    @pl.loop(0, PER_W * K // 16)
    def _(t):
        sl = pl.ds(t * 16, 16)
        nidx[sl] = xform(nidx[sl])

    sems = (sem0, sem1)

    def fire(j, s):
        jc = j * C
        sem = sems[s]
        cps = [pltpu.async_copy(center_hbm.at[cidx.at[pl.ds(jc, C)]],
                                crow.at[s], sem),
               pltpu.async_copy(context_hbm.at[xidx.at[pl.ds(jc, C)]],
                                xrow.at[s], sem)]
        for off, ln in ((0, 128), (128, 128), (256, 64)):
            cps.append(pltpu.async_copy(
                context_hbm.at[nidx.at[pl.ds(jc * K + off, ln)]],
                nrow.at[s, pl.ds(off, ln)], sem))
        return cps

    def compute(s):
        @plsc.parallel_loop(0, C, unroll=2)
        def _(e):
            c0 = crow[s, e, pl.ds(0, 16)]
            c1 = crow[s, e, pl.ds(16, 16)]
            c2 = crow[s, e, pl.ds(32, 16)]
            c3 = crow[s, e, pl.ds(48, 16)]
            x0 = xrow[s, e, pl.ds(0, 16)]
            x1 = xrow[s, e, pl.ds(16, 16)]
            x2 = xrow[s, e, pl.ds(32, 16)]
            x3 = xrow[s, e, pl.ds(48, 16)]
            pos = c0 * x0 + c1 * x1 + c2 * x2 + c3 * x3
            base_s = e * R * 16
            stage[pl.ds(base_s, 16)] = plsc.cumsum(-pos)
            for k in range(K):
                n0 = nrow[s, e * K + k, pl.ds(0, 16)]
                n1 = nrow[s, e * K + k, pl.ds(16, 16)]
                n2 = nrow[s, e * K + k, pl.ds(32, 16)]
                n3 = nrow[s, e * K + k, pl.ds(48, 16)]
                neg = n0 * c0 + n1 * c1 + n2 * c2 + n3 * c3
                stage[pl.ds(base_s + (1 + k) * 16, 16)] = plsc.cumsum(neg)

    lane = lax.iota(jnp.int32, 16)

    def epilogue(acc):
        @plsc.parallel_loop(0, R * C // 16, unroll=2, carry=acc)
        def acc_out(g, a):
            idx = lane * 16 + (g * 256 + 15)
            v = plsc.load_gather(stage, [idx])
            u = jnp.clip(v, -10.0, 10.0)
            return a + _softplus(u)
        return acc_out

    acc = jnp.zeros((16,), jnp.float32)
    prev = fire(0, 0)
    for j in range(NCHUNK):
        nxt = fire(j + 1, (j + 1) % 2) if j + 1 < NCHUNK else []
        for cp in prev:
            cp.wait()
        compute(j % 2)
        acc = epilogue(acc)
        prev = nxt

    accv[...] = acc
    pltpu.sync_copy(accv, out_hbm.at[wid])


@jax.jit
def _sc_loss(center_table, context_table, center_ids, context_ids, negf):
    centerp = _repack(center_table.T).reshape(2 * PACK, D)
    contextp = _repack(context_table.T).reshape(2 * PACK, D)
    mesh = plsc.VectorSubcoreMesh(core_axis_name="c", subcore_axis_name="s")
    f = pl.kernel(
        _body,
        out_type=jax.ShapeDtypeStruct((NW, 16), jnp.float32),
        mesh=mesh,
        compiler_params=pltpu.CompilerParams(
            needs_layout_passes=False, use_tc_tiling_on_sc=False),
        scratch_types=[
            pltpu.VMEM((PER_W,), jnp.int32),
            pltpu.VMEM((PER_W,), jnp.int32),
            pltpu.VMEM((K * PER_W,), jnp.int32),
            pltpu.VMEM((2, C, D), jnp.float32),
            pltpu.VMEM((2, C, D), jnp.float32),
            pltpu.VMEM((2, C * K, D), jnp.float32),
            pltpu.VMEM((R * C * 16,), jnp.float32),
            pltpu.VMEM((16,), jnp.float32),
            pltpu.SemaphoreType.DMA,
            pltpu.SemaphoreType.DMA,
        ],
    )
    return f(centerp, contextp, center_ids, context_ids, negf)


def kernel(center_table, context_table, center_ids, context_ids, neg_context_ids):
    negf = neg_context_ids.reshape(-1)
    partials = _sc_loss(center_table, context_table, center_ids,
                        context_ids, negf)
    return jnp.sum(partials) / B


# speedup vs baseline: 5.6234x; 1.0599x over previous
import jax
import jax.numpy as jnp
from jax import lax
from jax.experimental import pallas as pl
from jax.experimental.pallas import tpu as pltpu
from jax.experimental.pallas import tpu_sc as plsc

V = 1000000
D = 64
B = 16384
K = 5
NC = 2
NS = 16
NW = NC * NS
PER_W = B // NW
C = 64
NCHUNK = PER_W // C
R = 1 + K
TRC = 8192
NTR = (V + TRC - 1) // TRC


def _softplus(u):
    t = jnp.exp(-jnp.abs(u))
    s = t / (t + 2.0)
    p = s * s
    ser = s * (1.0 + p * (1.0 / 3.0 + p * (1.0 / 5.0 + p * (1.0 / 7.0 + p * (1.0 / 9.0)))))
    return jnp.maximum(u, 0.0) + 2.0 * ser


PACK = 1 << 19
TRC2 = 16384
NCB = (V + TRC2 - 1) // TRC2


def _mxu_t(x):
    return jnp.transpose(x)


def _tr_body(in1, in2, outb):
    outb[:, pl.ds(0, D)] = _mxu_t(in1[...])
    outb[:, pl.ds(D, D)] = _mxu_t(in2[...])


def _repack(tablet):
    return pl.pallas_call(
        _tr_body,
        grid=(PACK // TRC2,),
        in_specs=[
            pl.BlockSpec((D, TRC2), lambda j: (0, j)),
            pl.BlockSpec((D, TRC2),
                         lambda j: (0, jnp.minimum(j + PACK // TRC2, NCB - 1))),
        ],
        out_specs=pl.BlockSpec((TRC2, 2 * D), lambda j: (j, 0)),
        out_shape=jax.ShapeDtypeStruct((PACK, 2 * D), jnp.float32),
    )(tablet, tablet)


def _body(center_hbm, context_hbm, cids_hbm, xids_hbm, negf_hbm, out_hbm,
          cidx, xidx, nidx, crow, xrow, nrow, stage, accv, sem0, sem1):
    wid = lax.axis_index("s") * NC + lax.axis_index("c")
    base = wid * PER_W

    pltpu.sync_copy(cids_hbm.at[pl.ds(base, PER_W)], cidx)
    pltpu.sync_copy(xids_hbm.at[pl.ds(base, PER_W)], xidx)
    pltpu.sync_copy(negf_hbm.at[pl.ds(base * K, PER_W * K)], nidx)

    def xform(v):
        return ((v & (PACK - 1)) << 1) | jax.lax.shift_right_logical(v, 19)

    @pl.loop(0, PER_W // 16)
    def _(t):
        sl = pl.ds(t * 16, 16)
        cidx[sl] = xform(cidx[sl])
        xidx[sl] = xform(xidx[sl])

    @pl.loop(0, PER_W * K // 16)
    def _(t):
        sl = pl.ds(t * 16, 16)
        nidx[sl] = xform(nidx[sl])

    sems = (sem0, sem1)

    def fire(j, s):
        jc = j * C
        sem = sems[s]
        cps = [pltpu.async_copy(center_hbm.at[cidx.at[pl.ds(jc, C)]],
                                crow.at[s], sem),
               pltpu.async_copy(context_hbm.at[xidx.at[pl.ds(jc, C)]],
                                xrow.at[s], sem)]
        for off, ln in ((0, 128), (128, 128), (256, 64)):
            cps.append(pltpu.async_copy(
                context_hbm.at[nidx.at[pl.ds(jc * K + off, ln)]],
                nrow.at[s, pl.ds(off, ln)], sem))
        return cps

    def compute(s):
        @plsc.parallel_loop(0, C, unroll=2)
        def _(e):
            c0 = crow[s, e, pl.ds(0, 16)]
            c1 = crow[s, e, pl.ds(16, 16)]
            c2 = crow[s, e, pl.ds(32, 16)]
            c3 = crow[s, e, pl.ds(48, 16)]
            x0 = xrow[s, e, pl.ds(0, 16)]
            x1 = xrow[s, e, pl.ds(16, 16)]
            x2 = xrow[s, e, pl.ds(32, 16)]
            x3 = xrow[s, e, pl.ds(48, 16)]
            pos = c0 * x0 + c1 * x1 + c2 * x2 + c3 * x3
            base_s = e * R * 16
            stage[pl.ds(base_s, 16)] = plsc.cumsum(-pos)
            for k in range(K):
                n0 = nrow[s, e * K + k, pl.ds(0, 16)]
                n1 = nrow[s, e * K + k, pl.ds(16, 16)]
                n2 = nrow[s, e * K + k, pl.ds(32, 16)]
                n3 = nrow[s, e * K + k, pl.ds(48, 16)]
                neg = n0 * c0 + n1 * c1 + n2 * c2 + n3 * c3
                stage[pl.ds(base_s + (1 + k) * 16, 16)] = plsc.cumsum(neg)

    lane = lax.iota(jnp.int32, 16)

    def epilogue(acc):
        @plsc.parallel_loop(0, R * C // 16, unroll=2, carry=acc)
        def acc_out(g, a):
            idx = lane * 16 + (g * 256 + 15)
            v = plsc.load_gather(stage, [idx])
            u = jnp.clip(v, -10.0, 10.0)
            return a + _softplus(u)
        return acc_out

    acc = jnp.zeros((16,), jnp.float32)
    prev = fire(0, 0)
    for j in range(NCHUNK):
        nxt = fire(j + 1, (j + 1) % 2) if j + 1 < NCHUNK else []
        for cp in prev:
            cp.wait()
        compute(j % 2)
        acc = epilogue(acc)
        prev = nxt

    accv[...] = acc
    pltpu.sync_copy(accv, out_hbm.at[wid])


@jax.jit
def _sc_loss(center_table, context_table, center_ids, context_ids, negf):
    centerp = _repack(center_table.T).reshape(2 * PACK, D)
    contextp = _repack(context_table.T).reshape(2 * PACK, D)
    mesh = plsc.VectorSubcoreMesh(core_axis_name="c", subcore_axis_name="s")
    f = pl.kernel(
        _body,
        out_type=jax.ShapeDtypeStruct((NW, 16), jnp.float32),
        mesh=mesh,
        compiler_params=pltpu.CompilerParams(
            needs_layout_passes=False, use_tc_tiling_on_sc=False),
        scratch_types=[
            pltpu.VMEM((PER_W,), jnp.int32),
            pltpu.VMEM((PER_W,), jnp.int32),
            pltpu.VMEM((K * PER_W,), jnp.int32),
            pltpu.VMEM((2, C, D), jnp.float32),
            pltpu.VMEM((2, C, D), jnp.float32),
            pltpu.VMEM((2, C * K, D), jnp.float32),
            pltpu.VMEM((R * C * 16,), jnp.float32),
            pltpu.VMEM((16,), jnp.float32),
            pltpu.SemaphoreType.DMA,
            pltpu.SemaphoreType.DMA,
        ],
    )
    return f(centerp, contextp, center_ids, context_ids, negf)


def kernel(center_table, context_table, center_ids, context_ids, neg_context_ids):
    negf = neg_context_ids.reshape(-1)
    partials = _sc_loss(center_table, context_table, center_ids,
                        context_ids, negf)
    return jnp.sum(partials) / B
